# baseline (speedup 1.0000x reference)
import functools

import jax
import jax.numpy as jnp
from jax import lax
from jax.experimental import pallas as pl
from jax.experimental.pallas import tpu as pltpu
from jax.experimental.pallas import tpu_sc as plsc

BATCH = 1024
INPUT_SIZE = 256
NEURON_SIZE = 1024
EPS = 1e-6

_NC = 1
_NS = 16
_L = 16
_NW = _NC * _NS
_BPW = BATCH // _NW


def _tc_body(x_ref, w_ref, loc_ref, mean_ref, mdn_ref, locx_ref, locy_ref):
    x = x_ref[...] + EPS
    w = w_ref[...]
    g = jnp.dot(x, w, preferred_element_type=jnp.float32,
                precision=lax.Precision.HIGHEST)
    rn = jnp.sum(x * x, axis=1, keepdims=True)
    cn = jnp.sum(w * w, axis=0, keepdims=True)
    d = jnp.sqrt(rn + cn - 2.0 * g)
    dmin = jnp.min(d, axis=1, keepdims=True)
    idx = lax.broadcasted_iota(jnp.int32, d.shape, 1)
    mdn = jnp.min(jnp.where(d == dmin, idx, jnp.int32(NEURON_SIZE)),
                  axis=1)
    mean_ref[...] = jnp.sum(dmin).reshape(1, 1) * (1.0 / BATCH)
    mdn_ref[...] = mdn
    loc = loc_ref[...]
    locx_ref[...] = loc[:, 0]
    locy_ref[...] = loc[:, 1]


_tc_call = pl.pallas_call(
    _tc_body,
    out_shape=(
        jax.ShapeDtypeStruct((1, 1), jnp.float32),
        jax.ShapeDtypeStruct((BATCH,), jnp.int32),
        jax.ShapeDtypeStruct((NEURON_SIZE,), jnp.float32),
        jax.ShapeDtypeStruct((NEURON_SIZE,), jnp.float32),
    ),
)


def _sc_body(mdn_hbm, locx_hbm, locy_hbm, out_hbm, idx_v, locx_v, locy_v, out_v):
    wid = lax.axis_index("s") * _NC + lax.axis_index("c")
    base = wid * _BPW
    pltpu.sync_copy(locx_hbm, locx_v)
    pltpu.sync_copy(locy_hbm, locy_v)
    pltpu.sync_copy(mdn_hbm.at[pl.ds(base, _BPW)], idx_v)
    lane2 = lax.iota(jnp.int32, 16) * 2
    for i in range(_BPW // _L):
        idx = idx_v[pl.ds(i * _L, _L)]
        pos = lane2 + (2 * _L * i)
        plsc.store_scatter(out_v, [pos], plsc.load_gather(locx_v, [idx]))
        plsc.store_scatter(out_v, [pos + 1], plsc.load_gather(locy_v, [idx]))
    pltpu.sync_copy(out_v, out_hbm.at[pl.ds(base * 2, _BPW * 2)])


@functools.cache
def _sc_call():
    return pl.kernel(
        _sc_body,
        out_type=jax.ShapeDtypeStruct((BATCH * 2,), jnp.float32),
        mesh=plsc.VectorSubcoreMesh(core_axis_name="c", subcore_axis_name="s",
                                    num_cores=_NC, num_subcores=_NS),
        compiler_params=pltpu.CompilerParams(needs_layout_passes=False),
        scratch_types=[
            pltpu.VMEM((_BPW,), jnp.int32),
            pltpu.VMEM((NEURON_SIZE,), jnp.float32),
            pltpu.VMEM((NEURON_SIZE,), jnp.float32),
            pltpu.VMEM((_BPW * 2,), jnp.float32),
        ],
    )


def kernel(input, weight, location):
    mean_loss, mdn, locx, locy = _tc_call(input, weight, location)
    flat = _sc_call()(mdn, locx, locy)
    mdn_location = flat.reshape(BATCH, 1, 2)
    return jnp.reshape(mean_loss, ()), mdn_location


# speedup vs baseline: 1.0129x; 1.0129x over previous
import functools

import jax
import jax.numpy as jnp
from jax import lax
from jax.experimental import pallas as pl
from jax.experimental.pallas import tpu as pltpu
from jax.experimental.pallas import tpu_sc as plsc

BATCH = 1024
INPUT_SIZE = 256
NEURON_SIZE = 1024
EPS = 1e-6

_NC = 1
_NS = 16
_L = 16
_NW = _NC * _NS
_BPW = BATCH // _NW


_RB = 256
_GRID = BATCH // _RB


def _tc_body(x_ref, w_ref, mean_ref, mdn_ref):
    x = x_ref[...] + EPS
    w = w_ref[...]
    g = jnp.dot(x, w, preferred_element_type=jnp.float32,
                precision=lax.Precision.HIGHEST)
    rn = jnp.sum(x * x, axis=1, keepdims=True)
    cn = jnp.sum(w * w, axis=0, keepdims=True)
    d = jnp.sqrt(rn + cn - 2.0 * g)
    dmin = jnp.min(d, axis=1, keepdims=True)
    idx = lax.broadcasted_iota(jnp.int32, d.shape, 1)
    mdn = jnp.min(jnp.where(d == dmin, idx, jnp.int32(NEURON_SIZE)),
                  axis=1)
    psum = jnp.sum(dmin).reshape(1, 1) * (1.0 / BATCH)

    @pl.when(pl.program_id(0) == 0)
    def _():
        mean_ref[...] = jnp.zeros((1, 1), jnp.float32)

    mean_ref[...] += psum
    mdn_ref[...] = mdn


_tc_call = pl.pallas_call(
    _tc_body,
    grid=(_GRID,),
    in_specs=[
        pl.BlockSpec((_RB, INPUT_SIZE), lambda i: (i, 0)),
        pl.BlockSpec((INPUT_SIZE, NEURON_SIZE), lambda i: (0, 0)),
    ],
    out_specs=(
        pl.BlockSpec((1, 1), lambda i: (0, 0)),
        pl.BlockSpec((_RB,), lambda i: (i,)),
    ),
    out_shape=(
        jax.ShapeDtypeStruct((1, 1), jnp.float32),
        jax.ShapeDtypeStruct((BATCH,), jnp.int32),
    ),
)


def _sc_body(mdn_hbm, loc_hbm, out_hbm, idx_v, loc_v, out_v):
    wid = lax.axis_index("s") * _NC + lax.axis_index("c")
    base = wid * _BPW
    pltpu.sync_copy(loc_hbm, loc_v)
    pltpu.sync_copy(mdn_hbm.at[pl.ds(base, _BPW)], idx_v)
    lane2 = lax.iota(jnp.int32, 16) * 2
    for i in range(_BPW // _L):
        idx2 = idx_v[pl.ds(i * _L, _L)] * 2
        pos = lane2 + (2 * _L * i)
        plsc.store_scatter(out_v, [pos], plsc.load_gather(loc_v, [idx2]))
        plsc.store_scatter(out_v, [pos + 1], plsc.load_gather(loc_v, [idx2 + 1]))
    pltpu.sync_copy(out_v, out_hbm.at[pl.ds(base * 2, _BPW * 2)])


@functools.cache
def _sc_call():
    return pl.kernel(
        _sc_body,
        out_type=jax.ShapeDtypeStruct((BATCH * 2,), jnp.float32),
        mesh=plsc.VectorSubcoreMesh(core_axis_name="c", subcore_axis_name="s",
                                    num_cores=_NC, num_subcores=_NS),
        compiler_params=pltpu.CompilerParams(needs_layout_passes=False),
        scratch_types=[
            pltpu.VMEM((_BPW,), jnp.int32),
            pltpu.VMEM((NEURON_SIZE * 2,), jnp.float32),
            pltpu.VMEM((_BPW * 2,), jnp.float32),
        ],
    )


def kernel(input, weight, location):
    mean_loss, mdn = _tc_call(input, weight)
    flat = _sc_call()(mdn, location.reshape(NEURON_SIZE * 2))
    mdn_location = flat.reshape(BATCH, 1, 2)
    return jnp.reshape(mean_loss, ()), mdn_location


# speedup vs baseline: 1.0500x; 1.0366x over previous
import functools

import jax
import jax.numpy as jnp
from jax import lax
from jax.experimental import pallas as pl
from jax.experimental.pallas import tpu as pltpu
from jax.experimental.pallas import tpu_sc as plsc

BATCH = 1024
INPUT_SIZE = 256
NEURON_SIZE = 1024
EPS = 1e-6

_NC = 1
_NS = 16
_L = 16
_NW = _NC * _NS
_BPW = BATCH // _NW


def _tc_body(x_ref, w_ref, mean_ref, mdn_ref):
    x = x_ref[...] + EPS
    w = w_ref[...]
    g = jnp.dot(x, w, preferred_element_type=jnp.float32,
                precision=lax.Precision.HIGHEST)
    rn = jnp.sum(x * x, axis=1, keepdims=True)
    cn = jnp.sum(w * w, axis=0, keepdims=True)
    d = jnp.sqrt(rn + cn - 2.0 * g)
    dmin = jnp.min(d, axis=1, keepdims=True)
    mdn = jnp.argmin(d, axis=1)
    mean_ref[...] = jnp.sum(dmin).reshape(1, 1) * (1.0 / BATCH)
    mdn_ref[...] = mdn


_tc_call = pl.pallas_call(
    _tc_body,
    out_shape=(
        jax.ShapeDtypeStruct((1, 1), jnp.float32),
        jax.ShapeDtypeStruct((BATCH,), jnp.int32),
    ),
)


def _sc_body(mdn_hbm, loc_hbm, out_hbm, idx_v, loc_v, out_v):
    wid = lax.axis_index("s") * _NC + lax.axis_index("c")
    base = wid * _BPW
    pltpu.sync_copy(loc_hbm, loc_v)
    pltpu.sync_copy(mdn_hbm.at[pl.ds(base, _BPW)], idx_v)
    lane2 = lax.iota(jnp.int32, 16) * 2
    for i in range(_BPW // _L):
        idx2 = idx_v[pl.ds(i * _L, _L)] * 2
        pos = lane2 + (2 * _L * i)
        plsc.store_scatter(out_v, [pos], plsc.load_gather(loc_v, [idx2]))
        plsc.store_scatter(out_v, [pos + 1], plsc.load_gather(loc_v, [idx2 + 1]))
    pltpu.sync_copy(out_v, out_hbm.at[pl.ds(base * 2, _BPW * 2)])


@functools.cache
def _sc_call():
    return pl.kernel(
        _sc_body,
        out_type=jax.ShapeDtypeStruct((BATCH * 2,), jnp.float32),
        mesh=plsc.VectorSubcoreMesh(core_axis_name="c", subcore_axis_name="s",
                                    num_cores=_NC, num_subcores=_NS),
        compiler_params=pltpu.CompilerParams(needs_layout_passes=False),
        scratch_types=[
            pltpu.VMEM((_BPW,), jnp.int32),
            pltpu.VMEM((NEURON_SIZE * 2,), jnp.float32),
            pltpu.VMEM((_BPW * 2,), jnp.float32),
        ],
    )


def kernel(input, weight, location):
    mean_loss, mdn = _tc_call(input, weight)
    flat = _sc_call()(mdn, location.reshape(NEURON_SIZE * 2))
    mdn_location = flat.reshape(BATCH, 1, 2)
    return jnp.reshape(mean_loss, ()), mdn_location
